# baseline (speedup 1.0000x reference)
import jax
import jax.numpy as jnp
from jax.experimental import pallas as pl
from jax.experimental.pallas import tpu as pltpu

_BLOCK = 4096


def _dedicom_body(row_ref, col_ref, g_ref, lv_ref, out_ref):
    row = row_ref[...]
    col = col_ref[...]
    g = g_ref[...]
    k_rel = lv_ref.shape[0]
    recs = []
    for k in range(k_rel):
        dk = lv_ref[k, :]
        m_k = (dk[:, None] * g) * dk[None, :]
        left = jnp.dot(row, m_k,
                       preferred_element_type=jnp.float32)
        recs.append(jnp.sum(left * col, axis=1))
    scores = jnp.stack(recs, axis=0)
    out_ref[...] = jax.nn.sigmoid(scores)


def kernel(inputs_row, inputs_col, global_interaction, local_variation):
    n, d = inputs_row.shape
    k_rel = local_variation.shape[0]
    grid = (pl.cdiv(n, _BLOCK),)
    return pl.pallas_call(
        _dedicom_body,
        grid=grid,
        in_specs=[
            pl.BlockSpec((_BLOCK, d), lambda i: (i, 0)),
            pl.BlockSpec((_BLOCK, d), lambda i: (i, 0)),
            pl.BlockSpec((d, d), lambda i: (0, 0)),
            pl.BlockSpec((k_rel, d), lambda i: (0, 0)),
        ],
        out_specs=pl.BlockSpec((k_rel, _BLOCK), lambda i: (0, i)),
        out_shape=jax.ShapeDtypeStruct((k_rel, n), jnp.float32),
        compiler_params=pltpu.CompilerParams(
            dimension_semantics=("parallel",),
        ),
        name="dedicom_decoder",
    )(inputs_row, inputs_col, global_interaction, local_variation)


# speedup vs baseline: 1.1527x; 1.1527x over previous
import jax
import jax.numpy as jnp
from jax.experimental import pallas as pl
from jax.experimental.pallas import tpu as pltpu

_BLOCK = 4096


def _dedicom_body(row_ref, col_ref, g_ref, lv_ref, out_ref):
    rowb = row_ref[...].astype(jnp.bfloat16)
    colb = col_ref[...].astype(jnp.bfloat16)
    g = g_ref[...]
    lv = lv_ref[...]
    k_rel = lv.shape[0]
    d = g.shape[0]
    ts = []
    for k in range(k_rel):
        dk = lv[k]
        m_k = ((dk[:, None] * g) * dk[None, :]).astype(jnp.bfloat16)
        left = jnp.dot(rowb, m_k, preferred_element_type=jnp.float32)
        ts.append(left.astype(jnp.bfloat16) * colb)
    t_all = jnp.concatenate(ts, axis=1)
    m_idx = jax.lax.broadcasted_iota(jnp.int32, (k_rel * d, k_rel), 0)
    k_idx = jax.lax.broadcasted_iota(jnp.int32, (k_rel * d, k_rel), 1)
    seg = (m_idx // d == k_idx).astype(jnp.bfloat16)
    rec = jnp.dot(t_all, seg, preferred_element_type=jnp.float32)
    out_ref[...] = jax.nn.sigmoid(rec)


def kernel(inputs_row, inputs_col, global_interaction, local_variation):
    n, d = inputs_row.shape
    k_rel = local_variation.shape[0]
    grid = (pl.cdiv(n, _BLOCK),)
    out_nk = pl.pallas_call(
        _dedicom_body,
        grid=grid,
        in_specs=[
            pl.BlockSpec((_BLOCK, d), lambda i: (i, 0)),
            pl.BlockSpec((_BLOCK, d), lambda i: (i, 0)),
            pl.BlockSpec((d, d), lambda i: (0, 0)),
            pl.BlockSpec((k_rel, d), lambda i: (0, 0)),
        ],
        out_specs=pl.BlockSpec((_BLOCK, k_rel), lambda i: (i, 0)),
        out_shape=jax.ShapeDtypeStruct((n, k_rel), jnp.float32),
        compiler_params=pltpu.CompilerParams(
            dimension_semantics=("parallel",),
        ),
        name="dedicom_decoder",
    )(inputs_row, inputs_col, global_interaction, local_variation)
    return out_nk.T


# speedup vs baseline: 1.2340x; 1.0706x over previous
import jax
import jax.numpy as jnp
from jax.experimental import pallas as pl
from jax.experimental.pallas import tpu as pltpu

_BLOCK = 4096


def _dedicom_body(row_ref, col_ref, g_ref, lv_ref, out_ref):
    rowb = row_ref[...].astype(jnp.bfloat16)
    col = col_ref[...]
    g = g_ref[...]
    lv = lv_ref[...]
    k_rel = lv.shape[0]
    m_ks = [((lv[k][:, None] * g) * lv[k][None, :]).astype(jnp.bfloat16)
            for k in range(k_rel)]
    recs = []
    for k in range(k_rel):
        left = jnp.dot(rowb, m_ks[k], preferred_element_type=jnp.float32)
        recs.append(jnp.sum(left * col, axis=1))
    scores = jnp.stack(recs, axis=0)
    out_ref[...] = jax.nn.sigmoid(scores)


def kernel(inputs_row, inputs_col, global_interaction, local_variation):
    n, d = inputs_row.shape
    k_rel = local_variation.shape[0]
    grid = (pl.cdiv(n, _BLOCK),)
    out_nk = pl.pallas_call(
        _dedicom_body,
        grid=grid,
        in_specs=[
            pl.BlockSpec((_BLOCK, d), lambda i: (i, 0)),
            pl.BlockSpec((_BLOCK, d), lambda i: (i, 0)),
            pl.BlockSpec((d, d), lambda i: (0, 0)),
            pl.BlockSpec((k_rel, d), lambda i: (0, 0)),
        ],
        out_specs=pl.BlockSpec((k_rel, _BLOCK), lambda i: (0, i)),
        out_shape=jax.ShapeDtypeStruct((k_rel, n), jnp.float32),
        compiler_params=pltpu.CompilerParams(
            dimension_semantics=("parallel",),
        ),
        name="dedicom_decoder",
    )(inputs_row, inputs_col, global_interaction, local_variation)
    return out_nk


# speedup vs baseline: 1.3699x; 1.1101x over previous
import jax
import jax.numpy as jnp
from jax.experimental import pallas as pl
from jax.experimental.pallas import tpu as pltpu

_BLOCK = 8192


def _dedicom_body(row_ref, col_ref, g_ref, lv_ref, out_ref):
    row = row_ref[...]
    col = col_ref[...]
    g = g_ref[...]
    k_rel = lv_ref.shape[0]
    recs = []
    for k in range(k_rel):
        dk = lv_ref[k, :]
        left = jnp.dot(row * dk[None, :], g,
                       preferred_element_type=jnp.float32)
        recs.append(jnp.sum(left * (col * dk[None, :]), axis=1))
    scores = jnp.stack(recs, axis=0)
    out_ref[...] = jax.nn.sigmoid(scores)


def kernel(inputs_row, inputs_col, global_interaction, local_variation):
    n, d = inputs_row.shape
    k_rel = local_variation.shape[0]
    grid = (pl.cdiv(n, _BLOCK),)
    return pl.pallas_call(
        _dedicom_body,
        grid=grid,
        in_specs=[
            pl.BlockSpec((_BLOCK, d), lambda i: (i, 0)),
            pl.BlockSpec((_BLOCK, d), lambda i: (i, 0)),
            pl.BlockSpec((d, d), lambda i: (0, 0)),
            pl.BlockSpec((k_rel, d), lambda i: (0, 0)),
        ],
        out_specs=pl.BlockSpec((k_rel, _BLOCK), lambda i: (0, i)),
        out_shape=jax.ShapeDtypeStruct((k_rel, n), jnp.float32),
        compiler_params=pltpu.CompilerParams(
            dimension_semantics=("parallel",),
        ),
        name="dedicom_decoder",
    )(inputs_row, inputs_col, global_interaction, local_variation)
